# baseline (speedup 1.0000x reference)
import functools

import jax
import jax.numpy as jnp
from jax.experimental import pallas as pl
from jax.experimental.pallas import tpu as pltpu

T, C, H = 2048, 1024, 4096
E, TOPK = 8, 2
CAP = 320
STRIDE = 336
NSLOT = E * STRIDE
SENTINEL = CAP


def _dot(a, b):
    return jax.lax.dot_general(a, b, (((1,), (0,)), ((), ())),
                               preferred_element_type=jnp.float32)


def _mm_bias_kernel(a_ref, b_ref, bias_ref, o_ref, *, nsteps, relu):
    k = pl.program_id(2)
    part = _dot(a_ref[...], b_ref[...])

    @pl.when(k == 0)
    def _():
        o_ref[...] = part

    @pl.when(k > 0)
    def _():
        o_ref[...] += part

    @pl.when(k == nsteps - 1)
    def _():
        acc = o_ref[...] + bias_ref[...]
        o_ref[...] = jnp.maximum(acc, 0.0) if relu else acc


def _mm_bias(a, b, bias, relu, mt, nt, kt):
    M, K = a.shape
    _, N = b.shape
    grid = (M // mt, N // nt, K // kt)
    return pl.pallas_call(
        functools.partial(_mm_bias_kernel, nsteps=grid[2], relu=relu),
        grid=grid,
        in_specs=[
            pl.BlockSpec((mt, kt), lambda i, j, k: (i, k)),
            pl.BlockSpec((kt, nt), lambda i, j, k: (k, j)),
            pl.BlockSpec((1, nt), lambda i, j, k: (0, j)),
        ],
        out_specs=pl.BlockSpec((mt, nt), lambda i, j, k: (i, j)),
        out_shape=jax.ShapeDtypeStruct((M, N), jnp.float32),
        compiler_params=pltpu.CompilerParams(
            dimension_semantics=("parallel", "parallel", "arbitrary")),
    )(a, b, bias.reshape(1, -1))


def _dispatch_kernel(logits_ref, slot0_ref, slot1_ref, wslot_ref, cum_ref, a_ref):
    logits = logits_ref[...]
    lane = jax.lax.broadcasted_iota(jnp.int32, (T, E), 1)
    m = jnp.max(logits, axis=1, keepdims=True)
    ex = jnp.exp(logits - m)
    probs = ex / jnp.sum(ex, axis=1, keepdims=True)

    p0 = jnp.max(probs, axis=1, keepdims=True)
    e0 = jnp.min(jnp.where(probs == p0, lane, E), axis=1, keepdims=True)
    pm = jnp.where(lane == e0, -1.0, probs)
    p1 = jnp.max(pm, axis=1, keepdims=True)
    e1 = jnp.min(jnp.where(pm == p1, lane, E), axis=1, keepdims=True)

    oh0 = (lane == e0).astype(jnp.float32)
    oh1 = (lane == e1).astype(jnp.float32)
    a_ref[...] = oh0 + oh1

    row = jax.lax.broadcasted_iota(jnp.int32, (128, 128), 0)
    col = jax.lax.broadcasted_iota(jnp.int32, (128, 128), 1)
    lstrict = (col < row).astype(jnp.float32)

    def body(i, carry):
        ablk = a_ref[pl.ds(i * 128, 128), :]
        cum_ref[pl.ds(i * 128, 128), :] = carry + _dot(lstrict, ablk)
        return carry + jnp.sum(ablk, axis=0, keepdims=True)

    jax.lax.fori_loop(0, T // 128, body, jnp.zeros((1, E), jnp.float32))
    cum = cum_ref[...]

    r0 = jnp.sum(cum * oh0, axis=1, keepdims=True)
    r1 = jnp.sum(cum * oh1, axis=1, keepdims=True)
    kept0 = r0 < float(CAP)
    kept1 = r1 < float(CAP)
    fs0 = e0.astype(jnp.float32) * STRIDE + r0
    fs1 = e1.astype(jnp.float32) * STRIDE + r1
    s0 = jnp.where(kept0, fs0, float(SENTINEL)).astype(jnp.int32)
    s1 = jnp.where(kept1, fs1, float(SENTINEL)).astype(jnp.int32)
    w0 = jnp.where(kept0, p0, 0.0)
    w1 = jnp.where(kept1, p1, 0.0)
    slot0_ref[...] = s0
    slot1_ref[...] = s1

    def wbody(j, _):
        sidx = j * 128 + jax.lax.broadcasted_iota(jnp.int32, (T, 128), 1)
        m0 = jnp.where(s0 == sidx, w0, 0.0)
        m1 = jnp.where(s1 == sidx, w1, 0.0)
        wslot_ref[:, pl.ds(j * 128, 128)] = jnp.sum(m0 + m1, axis=0,
                                                    keepdims=True)
        return 0

    jax.lax.fori_loop(0, NSLOT // 128, wbody, 0)


def _dispatch(logits):
    return pl.pallas_call(
        _dispatch_kernel,
        in_specs=[pl.BlockSpec((T, E), lambda: (0, 0))],
        out_specs=[
            pl.BlockSpec((T, 1), lambda: (0, 0)),
            pl.BlockSpec((T, 1), lambda: (0, 0)),
            pl.BlockSpec((1, NSLOT), lambda: (0, 0)),
        ],
        out_shape=[
            jax.ShapeDtypeStruct((T, 1), jnp.int32),
            jax.ShapeDtypeStruct((T, 1), jnp.int32),
            jax.ShapeDtypeStruct((1, NSLOT), jnp.float32),
        ],
        scratch_shapes=[pltpu.VMEM((T, E), jnp.float32),
                        pltpu.VMEM((T, E), jnp.float32)],
    )(logits)


def _gather_kernel(s0_ref, s1_ref, x_ref, o_ref, *, nsteps, kt):
    i, k = pl.program_id(0), pl.program_id(2)
    rows = i * STRIDE + jax.lax.broadcasted_iota(jnp.int32, (STRIDE, kt), 0)
    s0 = s0_ref[...]
    s1 = s1_ref[...]
    sel = (s0 == rows).astype(jnp.float32) + (s1 == rows).astype(jnp.float32)
    part = _dot(sel, x_ref[...])

    @pl.when(k == 0)
    def _():
        o_ref[...] = part

    @pl.when(k > 0)
    def _():
        o_ref[...] += part


def _gather(s0t, s1t, x2, nt=512, kt=512):
    grid = (E, C // nt, T // kt)
    return pl.pallas_call(
        functools.partial(_gather_kernel, nsteps=grid[2], kt=kt),
        grid=grid,
        in_specs=[
            pl.BlockSpec((1, kt), lambda i, j, k: (0, k)),
            pl.BlockSpec((1, kt), lambda i, j, k: (0, k)),
            pl.BlockSpec((kt, nt), lambda i, j, k: (k, j)),
        ],
        out_specs=pl.BlockSpec((STRIDE, nt), lambda i, j, k: (i, j)),
        out_shape=jax.ShapeDtypeStruct((NSLOT, C), jnp.float32),
        compiler_params=pltpu.CompilerParams(
            dimension_semantics=("parallel", "parallel", "arbitrary")),
    )(s0t, s1t, x2)


def _ffn1_kernel(a_ref, w_ref, b_ref, o_ref, *, nsteps):
    k = pl.program_id(2)
    part = _dot(a_ref[...], w_ref[0])

    @pl.when(k == 0)
    def _():
        o_ref[...] = part

    @pl.when(k > 0)
    def _():
        o_ref[...] += part

    @pl.when(k == nsteps - 1)
    def _():
        o_ref[...] = jnp.maximum(o_ref[...] + b_ref[0], 0.0)


def _ffn1(xe, W1, b1, nt=512, kt=512):
    grid = (E, H // nt, C // kt)
    return pl.pallas_call(
        functools.partial(_ffn1_kernel, nsteps=grid[2]),
        grid=grid,
        in_specs=[
            pl.BlockSpec((STRIDE, kt), lambda e, j, k: (e, k)),
            pl.BlockSpec((1, kt, nt), lambda e, j, k: (e, k, j)),
            pl.BlockSpec((1, 1, nt), lambda e, j, k: (e, 0, j)),
        ],
        out_specs=pl.BlockSpec((STRIDE, nt), lambda e, j, k: (e, j)),
        out_shape=jax.ShapeDtypeStruct((NSLOT, H), jnp.float32),
        compiler_params=pltpu.CompilerParams(
            dimension_semantics=("parallel", "parallel", "arbitrary")),
    )(xe, W1, b1.reshape(E, 1, H))


def _ffn2_kernel(a_ref, w_ref, b_ref, ws_ref, o_ref, *, nsteps):
    k = pl.program_id(2)
    part = _dot(a_ref[...], w_ref[0])

    @pl.when(k == 0)
    def _():
        o_ref[...] = part

    @pl.when(k > 0)
    def _():
        o_ref[...] += part

    @pl.when(k == nsteps - 1)
    def _():
        o_ref[...] = (o_ref[...] + b_ref[0]) * ws_ref[...]


def _ffn2(he, W2, b2, wslot_col, nt=512, kt=512):
    grid = (E, C // nt, H // kt)
    return pl.pallas_call(
        functools.partial(_ffn2_kernel, nsteps=grid[2]),
        grid=grid,
        in_specs=[
            pl.BlockSpec((STRIDE, kt), lambda e, j, k: (e, k)),
            pl.BlockSpec((1, kt, nt), lambda e, j, k: (e, k, j)),
            pl.BlockSpec((1, 1, nt), lambda e, j, k: (e, 0, j)),
            pl.BlockSpec((STRIDE, 1), lambda e, j, k: (e, 0)),
        ],
        out_specs=pl.BlockSpec((STRIDE, nt), lambda e, j, k: (e, j)),
        out_shape=jax.ShapeDtypeStruct((NSLOT, C), jnp.float32),
        compiler_params=pltpu.CompilerParams(
            dimension_semantics=("parallel", "parallel", "arbitrary")),
    )(he, W2, b2.reshape(E, 1, C), wslot_col)


def _combine_kernel(s0_ref, s1_ref, y_ref, o_ref, *, nsteps, mt, kt):
    k = pl.program_id(2)
    scol = k * kt + jax.lax.broadcasted_iota(jnp.int32, (mt, kt), 1)
    sel = ((s0_ref[...] == scol).astype(jnp.float32)
           + (s1_ref[...] == scol).astype(jnp.float32))
    part = _dot(sel, y_ref[...])

    @pl.when(k == 0)
    def _():
        o_ref[...] = part

    @pl.when(k > 0)
    def _():
        o_ref[...] += part


def _combine(s0, s1, Y, mt=256, nt=512, kt=336):
    grid = (T // mt, C // nt, NSLOT // kt)
    return pl.pallas_call(
        functools.partial(_combine_kernel, nsteps=grid[2], mt=mt, kt=kt),
        grid=grid,
        in_specs=[
            pl.BlockSpec((mt, 1), lambda i, j, k: (i, 0)),
            pl.BlockSpec((mt, 1), lambda i, j, k: (i, 0)),
            pl.BlockSpec((kt, nt), lambda i, j, k: (k, j)),
        ],
        out_specs=pl.BlockSpec((mt, nt), lambda i, j, k: (i, j)),
        out_shape=jax.ShapeDtypeStruct((T, C), jnp.float32),
        compiler_params=pltpu.CompilerParams(
            dimension_semantics=("parallel", "parallel", "arbitrary")),
    )(s0, s1, Y)


def kernel(x, Wr1, br1, Wr2, br2, Wr3, br3, W1, b1, W2, b2):
    x2 = x.reshape(T, C)
    h1 = _mm_bias(x2, Wr1, br1, True, 256, 512, 512)
    h2 = _mm_bias(h1, Wr2, br2, True, 256, 512, 512)
    logits = _mm_bias(h2, Wr3, br3, False, 256, 8, 512)
    slot0, slot1, wslot = _dispatch(logits)
    s0t = slot0.reshape(1, T)
    s1t = slot1.reshape(1, T)
    xe = _gather(s0t, s1t, x2)
    he = _ffn1(xe, W1, b1)
    Y = _ffn2(he, W2, b2, wslot.reshape(NSLOT, 1))
    out = _combine(slot0, slot1, Y)
    return out.reshape(1, T, C)


# speedup vs baseline: 3.4995x; 3.4995x over previous
import functools

import jax
import jax.numpy as jnp
from jax.experimental import pallas as pl
from jax.experimental.pallas import tpu as pltpu

T, C, H = 2048, 1024, 4096
E, TOPK = 8, 2
CAP = 320
STRIDE = 336
NSLOT = E * STRIDE
SENTINEL = CAP


def _dot(a, b):
    return jax.lax.dot_general(a, b, (((1,), (0,)), ((), ())),
                               preferred_element_type=jnp.float32)


def _mm_kernel(a_ref, b_ref, bias_ref, o_ref, *, relu):
    acc = _dot(a_ref[...], b_ref[...]) + bias_ref[...]
    o_ref[...] = jnp.maximum(acc, 0.0) if relu else acc


def _mm_resident(a, b, bias, relu, nt):
    M, K = a.shape
    _, N = b.shape
    return pl.pallas_call(
        functools.partial(_mm_kernel, relu=relu),
        grid=(N // nt,),
        in_specs=[
            pl.BlockSpec((M, K), lambda j: (0, 0)),
            pl.BlockSpec((K, nt), lambda j: (0, j)),
            pl.BlockSpec((1, nt), lambda j: (0, j)),
        ],
        out_specs=pl.BlockSpec((M, nt), lambda j: (0, j)),
        out_shape=jax.ShapeDtypeStruct((M, N), jnp.float32),
        compiler_params=pltpu.CompilerParams(
            dimension_semantics=("arbitrary",)),
    )(a, b, bias.reshape(1, -1))


def _mm_stream_a(a, b, bias, relu, mt):
    M, K = a.shape
    _, N = b.shape
    return pl.pallas_call(
        functools.partial(_mm_kernel, relu=relu),
        grid=(M // mt,),
        in_specs=[
            pl.BlockSpec((mt, K), lambda i: (i, 0)),
            pl.BlockSpec((K, N), lambda i: (0, 0)),
            pl.BlockSpec((1, N), lambda i: (0, 0)),
        ],
        out_specs=pl.BlockSpec((mt, N), lambda i: (i, 0)),
        out_shape=jax.ShapeDtypeStruct((M, N), jnp.float32),
        compiler_params=pltpu.CompilerParams(
            dimension_semantics=("arbitrary",)),
    )(a, b, bias.reshape(1, -1))


def _dispatch_kernel(logits_ref, slot0_ref, slot1_ref, wslot_ref, cum_ref, a_ref):
    logits = logits_ref[...]
    lane = jax.lax.broadcasted_iota(jnp.int32, (T, E), 1)
    m = jnp.max(logits, axis=1, keepdims=True)
    ex = jnp.exp(logits - m)
    probs = ex / jnp.sum(ex, axis=1, keepdims=True)

    p0 = jnp.max(probs, axis=1, keepdims=True)
    e0 = jnp.min(jnp.where(probs == p0, lane, E), axis=1, keepdims=True)
    pm = jnp.where(lane == e0, -1.0, probs)
    p1 = jnp.max(pm, axis=1, keepdims=True)
    e1 = jnp.min(jnp.where(pm == p1, lane, E), axis=1, keepdims=True)

    oh0 = (lane == e0).astype(jnp.float32)
    oh1 = (lane == e1).astype(jnp.float32)
    a_ref[...] = oh0 + oh1

    row = jax.lax.broadcasted_iota(jnp.int32, (128, 128), 0)
    col = jax.lax.broadcasted_iota(jnp.int32, (128, 128), 1)
    lstrict = (col < row).astype(jnp.float32)

    def body(i, carry):
        ablk = a_ref[pl.ds(i * 128, 128), :]
        cum_ref[pl.ds(i * 128, 128), :] = carry + _dot(lstrict, ablk)
        return carry + jnp.sum(ablk, axis=0, keepdims=True)

    jax.lax.fori_loop(0, T // 128, body, jnp.zeros((1, E), jnp.float32))
    cum = cum_ref[...]

    r0 = jnp.sum(cum * oh0, axis=1, keepdims=True)
    r1 = jnp.sum(cum * oh1, axis=1, keepdims=True)
    kept0 = r0 < float(CAP)
    kept1 = r1 < float(CAP)
    fs0 = e0.astype(jnp.float32) * STRIDE + r0
    fs1 = e1.astype(jnp.float32) * STRIDE + r1
    s0 = jnp.where(kept0, fs0, float(SENTINEL)).astype(jnp.int32)
    s1 = jnp.where(kept1, fs1, float(SENTINEL)).astype(jnp.int32)
    w0 = jnp.where(kept0, p0, 0.0)
    w1 = jnp.where(kept1, p1, 0.0)
    slot0_ref[...] = s0
    slot1_ref[...] = s1

    def wbody(j, _):
        sidx = j * 128 + jax.lax.broadcasted_iota(jnp.int32, (T, 128), 1)
        m0 = jnp.where(s0 == sidx, w0, 0.0)
        m1 = jnp.where(s1 == sidx, w1, 0.0)
        wslot_ref[:, pl.ds(j * 128, 128)] = jnp.sum(m0 + m1, axis=0,
                                                    keepdims=True)
        return 0

    jax.lax.fori_loop(0, NSLOT // 128, wbody, 0)


def _dispatch(logits):
    return pl.pallas_call(
        _dispatch_kernel,
        in_specs=[pl.BlockSpec((T, E), lambda: (0, 0))],
        out_specs=[
            pl.BlockSpec((T, 1), lambda: (0, 0)),
            pl.BlockSpec((T, 1), lambda: (0, 0)),
            pl.BlockSpec((1, NSLOT), lambda: (0, 0)),
        ],
        out_shape=[
            jax.ShapeDtypeStruct((T, 1), jnp.int32),
            jax.ShapeDtypeStruct((T, 1), jnp.int32),
            jax.ShapeDtypeStruct((1, NSLOT), jnp.float32),
        ],
        scratch_shapes=[pltpu.VMEM((T, E), jnp.float32),
                        pltpu.VMEM((T, E), jnp.float32)],
    )(logits)


def _gather_kernel(s0_ref, s1_ref, x_ref, o_ref):
    e = pl.program_id(0)
    rows = e * STRIDE + jax.lax.broadcasted_iota(jnp.int32, (STRIDE, T), 0)
    sel = ((s0_ref[...] == rows).astype(jnp.float32)
           + (s1_ref[...] == rows).astype(jnp.float32))
    o_ref[...] = _dot(sel, x_ref[...])


def _gather(s0t, s1t, x2):
    return pl.pallas_call(
        _gather_kernel,
        grid=(E,),
        in_specs=[
            pl.BlockSpec((1, T), lambda e: (0, 0)),
            pl.BlockSpec((1, T), lambda e: (0, 0)),
            pl.BlockSpec((T, C), lambda e: (0, 0)),
        ],
        out_specs=pl.BlockSpec((STRIDE, C), lambda e: (e, 0)),
        out_shape=jax.ShapeDtypeStruct((NSLOT, C), jnp.float32),
        compiler_params=pltpu.CompilerParams(
            dimension_semantics=("arbitrary",)),
    )(s0t, s1t, x2)


def _ffn1_kernel(a_ref, w_ref, b_ref, o_ref):
    o_ref[...] = jnp.maximum(_dot(a_ref[...], w_ref[0]) + b_ref[0], 0.0)


def _ffn1(xe, W1, b1, nt=512):
    grid = (E, H // nt)
    return pl.pallas_call(
        _ffn1_kernel,
        grid=grid,
        in_specs=[
            pl.BlockSpec((STRIDE, C), lambda e, j: (e, 0)),
            pl.BlockSpec((1, C, nt), lambda e, j: (e, 0, j)),
            pl.BlockSpec((1, 1, nt), lambda e, j: (e, 0, j)),
        ],
        out_specs=pl.BlockSpec((STRIDE, nt), lambda e, j: (e, j)),
        out_shape=jax.ShapeDtypeStruct((NSLOT, H), jnp.float32),
        compiler_params=pltpu.CompilerParams(
            dimension_semantics=("arbitrary", "arbitrary")),
    )(xe, W1, b1.reshape(E, 1, H))


def _ffn2_kernel(a_ref, w_ref, b_ref, ws_ref, o_ref):
    o_ref[...] = (_dot(a_ref[...], w_ref[0]) + b_ref[0]) * ws_ref[...]


def _ffn2(he, W2, b2, wslot_col, nt=512):
    grid = (E, C // nt)
    return pl.pallas_call(
        _ffn2_kernel,
        grid=grid,
        in_specs=[
            pl.BlockSpec((STRIDE, H), lambda e, j: (e, 0)),
            pl.BlockSpec((1, H, nt), lambda e, j: (e, 0, j)),
            pl.BlockSpec((1, 1, nt), lambda e, j: (e, 0, j)),
            pl.BlockSpec((STRIDE, 1), lambda e, j: (e, 0)),
        ],
        out_specs=pl.BlockSpec((STRIDE, nt), lambda e, j: (e, j)),
        out_shape=jax.ShapeDtypeStruct((NSLOT, C), jnp.float32),
        compiler_params=pltpu.CompilerParams(
            dimension_semantics=("arbitrary", "arbitrary")),
    )(he, W2, b2.reshape(E, 1, C), wslot_col)


def _combine_kernel(s0_ref, s1_ref, y_ref, o_ref, *, mt):
    scol = jax.lax.broadcasted_iota(jnp.int32, (mt, NSLOT), 1)
    sel = ((s0_ref[...] == scol).astype(jnp.float32)
           + (s1_ref[...] == scol).astype(jnp.float32))
    o_ref[...] = _dot(sel, y_ref[...])


def _combine(s0, s1, Y, mt=256):
    return pl.pallas_call(
        functools.partial(_combine_kernel, mt=mt),
        grid=(T // mt,),
        in_specs=[
            pl.BlockSpec((mt, 1), lambda i: (i, 0)),
            pl.BlockSpec((mt, 1), lambda i: (i, 0)),
            pl.BlockSpec((NSLOT, C), lambda i: (0, 0)),
        ],
        out_specs=pl.BlockSpec((mt, C), lambda i: (i, 0)),
        out_shape=jax.ShapeDtypeStruct((T, C), jnp.float32),
        compiler_params=pltpu.CompilerParams(
            dimension_semantics=("arbitrary",)),
    )(s0, s1, Y)


def kernel(x, Wr1, br1, Wr2, br2, Wr3, br3, W1, b1, W2, b2):
    x2 = x.reshape(T, C)
    h1 = _mm_resident(x2, Wr1, br1, True, 512)
    h2 = _mm_resident(h1, Wr2, br2, True, 256)
    logits = _mm_stream_a(h2, Wr3, br3, False, 256)
    slot0, slot1, wslot = _dispatch(logits)
    s0t = slot0.reshape(1, T)
    s1t = slot1.reshape(1, T)
    xe = _gather(s0t, s1t, x2)
    he = _ffn1(xe, W1, b1)
    Y = _ffn2(he, W2, b2, wslot.reshape(NSLOT, 1))
    out = _combine(slot0, slot1, Y)
    return out.reshape(1, T, C)


# speedup vs baseline: 4.2848x; 1.2244x over previous
import functools

import jax
import jax.numpy as jnp
from jax.experimental import pallas as pl
from jax.experimental.pallas import tpu as pltpu

T, C, H = 2048, 1024, 4096
E, TOPK = 8, 2
CAP = 320
STRIDE = 336
NSLOT = E * STRIDE
SENTINEL = CAP


def _dot(a, b):
    return jax.lax.dot_general(a, b, (((1,), (0,)), ((), ())),
                               preferred_element_type=jnp.float32)


def _mm_kernel(a_ref, b_ref, bias_ref, o_ref, *, relu):
    acc = _dot(a_ref[...], b_ref[...]) + bias_ref[...]
    o_ref[...] = jnp.maximum(acc, 0.0) if relu else acc


def _mm_resident(a, b, bias, relu, nt):
    M, K = a.shape
    _, N = b.shape
    return pl.pallas_call(
        functools.partial(_mm_kernel, relu=relu),
        grid=(N // nt,),
        in_specs=[
            pl.BlockSpec((M, K), lambda j: (0, 0)),
            pl.BlockSpec((K, nt), lambda j: (0, j)),
            pl.BlockSpec((1, nt), lambda j: (0, j)),
        ],
        out_specs=pl.BlockSpec((M, nt), lambda j: (0, j)),
        out_shape=jax.ShapeDtypeStruct((M, N), jnp.float32),
        compiler_params=pltpu.CompilerParams(
            dimension_semantics=("arbitrary",)),
    )(a, b, bias.reshape(1, -1))


def _mm_stream_a(a, b, bias, relu, mt):
    M, K = a.shape
    _, N = b.shape
    return pl.pallas_call(
        functools.partial(_mm_kernel, relu=relu),
        grid=(M // mt,),
        in_specs=[
            pl.BlockSpec((mt, K), lambda i: (i, 0)),
            pl.BlockSpec((K, N), lambda i: (0, 0)),
            pl.BlockSpec((1, N), lambda i: (0, 0)),
        ],
        out_specs=pl.BlockSpec((mt, N), lambda i: (i, 0)),
        out_shape=jax.ShapeDtypeStruct((M, N), jnp.float32),
        compiler_params=pltpu.CompilerParams(
            dimension_semantics=("arbitrary",)),
    )(a, b, bias.reshape(1, -1))


def _dispatch_body(logits, slot0_ref, slot1_ref, wslot_ref, cum_ref, a_ref):
    lane = jax.lax.broadcasted_iota(jnp.int32, (T, E), 1)
    m = jnp.max(logits, axis=1, keepdims=True)
    ex = jnp.exp(logits - m)
    probs = ex / jnp.sum(ex, axis=1, keepdims=True)

    p0 = jnp.max(probs, axis=1, keepdims=True)
    e0 = jnp.min(jnp.where(probs == p0, lane, E), axis=1, keepdims=True)
    pm = jnp.where(lane == e0, -1.0, probs)
    p1 = jnp.max(pm, axis=1, keepdims=True)
    e1 = jnp.min(jnp.where(pm == p1, lane, E), axis=1, keepdims=True)

    oh0 = (lane == e0).astype(jnp.float32)
    oh1 = (lane == e1).astype(jnp.float32)
    a_ref[...] = oh0 + oh1

    row = jax.lax.broadcasted_iota(jnp.int32, (128, 128), 0)
    col = jax.lax.broadcasted_iota(jnp.int32, (128, 128), 1)
    lstrict = (col < row).astype(jnp.float32)

    def body(i, carry):
        ablk = a_ref[pl.ds(i * 128, 128), :]
        cum_ref[pl.ds(i * 128, 128), :] = carry + _dot(lstrict, ablk)
        return carry + jnp.sum(ablk, axis=0, keepdims=True)

    jax.lax.fori_loop(0, T // 128, body, jnp.zeros((1, E), jnp.float32))
    cum = cum_ref[...]

    r0 = jnp.sum(cum * oh0, axis=1, keepdims=True)
    r1 = jnp.sum(cum * oh1, axis=1, keepdims=True)
    kept0 = r0 < float(CAP)
    kept1 = r1 < float(CAP)
    fs0 = e0.astype(jnp.float32) * STRIDE + r0
    fs1 = e1.astype(jnp.float32) * STRIDE + r1
    s0 = jnp.where(kept0, fs0, float(SENTINEL)).astype(jnp.int32)
    s1 = jnp.where(kept1, fs1, float(SENTINEL)).astype(jnp.int32)
    w0 = jnp.where(kept0, p0, 0.0)
    w1 = jnp.where(kept1, p1, 0.0)
    slot0_ref[...] = s0
    slot1_ref[...] = s1

    def wbody(j, _):
        sidx = j * 128 + jax.lax.broadcasted_iota(jnp.int32, (T, 128), 1)
        m0 = jnp.where(s0 == sidx, w0, 0.0)
        m1 = jnp.where(s1 == sidx, w1, 0.0)
        wslot_ref[:, pl.ds(j * 128, 128)] = jnp.sum(m0 + m1, axis=0,
                                                    keepdims=True)
        return 0

    jax.lax.fori_loop(0, NSLOT // 128, wbody, 0)


def _dispatch_kernel(logits_ref, slot0_ref, slot1_ref, wslot_ref,
                     cum_ref, a_ref):
    _dispatch_body(logits_ref[...], slot0_ref, slot1_ref, wslot_ref,
                   cum_ref, a_ref)


def _dispatch(logits):
    return pl.pallas_call(
        _dispatch_kernel,
        in_specs=[pl.BlockSpec((T, E), lambda: (0, 0))],
        out_specs=[
            pl.BlockSpec((T, 1), lambda: (0, 0)),
            pl.BlockSpec((T, 1), lambda: (0, 0)),
            pl.BlockSpec((1, NSLOT), lambda: (0, 0)),
        ],
        out_shape=[
            jax.ShapeDtypeStruct((T, 1), jnp.int32),
            jax.ShapeDtypeStruct((T, 1), jnp.int32),
            jax.ShapeDtypeStruct((1, NSLOT), jnp.float32),
        ],
        scratch_shapes=[pltpu.VMEM((T, E), jnp.float32),
                        pltpu.VMEM((T, E), jnp.float32)],
    )(logits)


def _mm2_kernel(a_ref, b_ref, bias_ref, wr3_ref, br3_ref, lg_ref, *, nsteps):
    j = pl.program_id(0)
    h2 = jnp.maximum(_dot(a_ref[...], b_ref[...]) + bias_ref[...], 0.0)
    part = _dot(h2, wr3_ref[...])

    @pl.when(j == 0)
    def _():
        lg_ref[...] = part + br3_ref[...]

    @pl.when(j > 0)
    def _():
        lg_ref[...] += part


def _mm2_logits(h1, Wr2, br2, Wr3, br3, nt=256):
    grid = (H // nt,)
    return pl.pallas_call(
        functools.partial(_mm2_kernel, nsteps=grid[0]),
        grid=grid,
        in_specs=[
            pl.BlockSpec((T, H), lambda j: (0, 0)),
            pl.BlockSpec((H, nt), lambda j: (0, j)),
            pl.BlockSpec((1, nt), lambda j: (0, j)),
            pl.BlockSpec((nt, E), lambda j: (j, 0)),
            pl.BlockSpec((1, E), lambda j: (0, 0)),
        ],
        out_specs=pl.BlockSpec((T, E), lambda j: (0, 0)),
        out_shape=jax.ShapeDtypeStruct((T, E), jnp.float32),
        compiler_params=pltpu.CompilerParams(
            dimension_semantics=("arbitrary",)),
    )(h1, Wr2, br2.reshape(1, -1), Wr3, br3.reshape(1, -1))


def _gather_kernel(s0_ref, s1_ref, x_ref, o_ref):
    e = pl.program_id(0)
    rows = e * STRIDE + jax.lax.broadcasted_iota(jnp.int32, (STRIDE, T), 0)
    sel = ((s0_ref[...] == rows).astype(jnp.float32)
           + (s1_ref[...] == rows).astype(jnp.float32))
    o_ref[...] = _dot(sel, x_ref[...])


def _gather(s0t, s1t, x2):
    return pl.pallas_call(
        _gather_kernel,
        grid=(E,),
        in_specs=[
            pl.BlockSpec((1, T), lambda e: (0, 0)),
            pl.BlockSpec((1, T), lambda e: (0, 0)),
            pl.BlockSpec((T, C), lambda e: (0, 0)),
        ],
        out_specs=pl.BlockSpec((STRIDE, C), lambda e: (e, 0)),
        out_shape=jax.ShapeDtypeStruct((NSLOT, C), jnp.float32),
        compiler_params=pltpu.CompilerParams(
            dimension_semantics=("arbitrary",)),
    )(s0t, s1t, x2)


def _ffn_kernel(xe_ref, w1_ref, b1_ref, w2_ref, b2_ref, ws_ref, o_ref,
                *, nsteps):
    j = pl.program_id(1)
    hblk = jnp.maximum(_dot(xe_ref[...], w1_ref[0]) + b1_ref[0], 0.0)
    part = _dot(hblk, w2_ref[0])

    @pl.when(j == 0)
    def _():
        o_ref[...] = part

    @pl.when(j > 0)
    def _():
        o_ref[...] += part

    @pl.when(j == nsteps - 1)
    def _():
        o_ref[...] = (o_ref[...] + b2_ref[0]) * ws_ref[...]


def _ffn(xe, W1, b1, W2, b2, wslot_col, ht=1024):
    grid = (E, H // ht)
    return pl.pallas_call(
        functools.partial(_ffn_kernel, nsteps=grid[1]),
        grid=grid,
        in_specs=[
            pl.BlockSpec((STRIDE, C), lambda e, j: (e, 0)),
            pl.BlockSpec((1, C, ht), lambda e, j: (e, 0, j)),
            pl.BlockSpec((1, 1, ht), lambda e, j: (e, 0, j)),
            pl.BlockSpec((1, ht, C), lambda e, j: (e, j, 0)),
            pl.BlockSpec((1, 1, C), lambda e, j: (e, 0, 0)),
            pl.BlockSpec((STRIDE, 1), lambda e, j: (e, 0)),
        ],
        out_specs=pl.BlockSpec((STRIDE, C), lambda e, j: (e, 0)),
        out_shape=jax.ShapeDtypeStruct((NSLOT, C), jnp.float32),
        compiler_params=pltpu.CompilerParams(
            dimension_semantics=("arbitrary", "arbitrary")),
    )(xe, W1, b1.reshape(E, 1, H), W2, b2.reshape(E, 1, C), wslot_col)


def _combine_kernel(s0_ref, s1_ref, y_ref, o_ref, *, mt):
    scol = jax.lax.broadcasted_iota(jnp.int32, (mt, NSLOT), 1)
    sel = ((s0_ref[...] == scol).astype(jnp.float32)
           + (s1_ref[...] == scol).astype(jnp.float32))
    o_ref[...] = _dot(sel, y_ref[...])


def _combine(s0, s1, Y, mt=256):
    return pl.pallas_call(
        functools.partial(_combine_kernel, mt=mt),
        grid=(T // mt,),
        in_specs=[
            pl.BlockSpec((mt, 1), lambda i: (i, 0)),
            pl.BlockSpec((mt, 1), lambda i: (i, 0)),
            pl.BlockSpec((NSLOT, C), lambda i: (0, 0)),
        ],
        out_specs=pl.BlockSpec((mt, C), lambda i: (i, 0)),
        out_shape=jax.ShapeDtypeStruct((T, C), jnp.float32),
        compiler_params=pltpu.CompilerParams(
            dimension_semantics=("arbitrary",)),
    )(s0, s1, Y)


def kernel(x, Wr1, br1, Wr2, br2, Wr3, br3, W1, b1, W2, b2):
    x2 = x.reshape(T, C)
    h1 = _mm_resident(x2, Wr1, br1, True, 512)
    logits = _mm2_logits(h1, Wr2, br2, Wr3, br3)
    slot0, slot1, wslot = _dispatch(logits)
    s0t = slot0.reshape(1, T)
    s1t = slot1.reshape(1, T)
    xe = _gather(s0t, s1t, x2)
    Y = _ffn(xe, W1, b1, W2, b2, wslot.reshape(NSLOT, 1))
    out = _combine(slot0, slot1, Y)
    return out.reshape(1, T, C)
